# baseline (speedup 1.0000x reference)
import functools

import jax
import jax.numpy as jnp
from jax import lax
from jax.experimental import pallas as pl
from jax.experimental.pallas import tpu as pltpu
from jax.experimental.pallas import tpu_sc as plsc

_B, _H, _W, _C = 4, 384, 384, 96
_N = _B * _H * _W
_HW = _H * _W
_K = 128
_G = _K // 16


def _floorf(x):
    xi = x.astype(jnp.int32)
    xf = xi.astype(jnp.float32)
    return jnp.where(xf > x, xf - 1.0, xf)


def _coords(g, extent):
    t = ((g + 1.0) * float(extent) - 1.0) * 0.5
    t1 = _floorf(t)
    f = t - t1
    snap = (1.0 - f) < 1e-5
    t1 = jnp.where(snap, t1 + 1.0, t1)
    f = jnp.where(snap, 0.0, f)
    return t1.astype(jnp.int32), f


def _sc_body(table, gxr, gyr, outr,
             gxv, gyv, ia, ib, ic, id_, war, wbr, wcr, wdr,
             ra, rb, rc, rd, ov, sem):
    nc = lax.axis_size("c")
    wid = lax.axis_index("s") * nc + lax.axis_index("c")
    pw = _N // (nc * lax.axis_size("s"))
    base = wid * pw
    brow = (base // _HW) * _HW
    iota = lax.iota(jnp.int32, 16)

    def chunk(t, carry):
        s0 = base + t * _K
        pltpu.sync_copy(gxr.at[pl.ds(s0, _K)], gxv)
        pltpu.sync_copy(gyr.at[pl.ds(s0, _K)], gyv)
        for g in range(_G):
            sl = pl.ds(g * 16, 16)
            x1, fx = _coords(gxv[sl], _W)
            y1, fy = _coords(gyv[sl], _H)
            x2 = x1 + 1
            y2 = y1 + 1
            vx1 = (x1 >= 0) & (x1 < _W)
            vx2 = (x2 >= 0) & (x2 < _W)
            vy1 = (y1 >= 0) & (y1 < _H)
            vy2 = (y2 >= 0) & (y2 < _H)
            cx1 = jnp.minimum(jnp.maximum(x1, 0), _W - 1)
            cx2 = jnp.minimum(jnp.maximum(x2, 0), _W - 1)
            cy1 = jnp.minimum(jnp.maximum(y1, 0), _H - 1)
            cy2 = jnp.minimum(jnp.maximum(y2, 0), _H - 1)
            ia[sl] = brow + cy1 * _W + cx1
            ib[sl] = brow + cy2 * _W + cx1
            ic[sl] = brow + cy1 * _W + cx2
            id_[sl] = brow + cy2 * _W + cx2
            gx1 = 1.0 - fx
            gy1 = 1.0 - fy
            war[sl] = jnp.where(vx1 & vy1, gx1 * gy1, 0.0)
            wbr[sl] = jnp.where(vx1 & vy2, gx1 * fy, 0.0)
            wcr[sl] = jnp.where(vx2 & vy1, fx * gy1, 0.0)
            wdr[sl] = jnp.where(vx2 & vy2, fx * fy, 0.0)
        da = pltpu.async_copy(table.at[ia], ra, sem)
        db = pltpu.async_copy(table.at[ib], rb, sem)
        dc = pltpu.async_copy(table.at[ic], rc, sem)
        dd = pltpu.async_copy(table.at[id_], rd, sem)
        da.wait()
        db.wait()
        dc.wait()
        dd.wait()
        for g in range(_G):
            sl = pl.ds(g * 16, 16)
            samp = iota + g * 16
            wav = war[sl]
            wbv = wbr[sl]
            wcv = wcr[sl]
            wdv = wdr[sl]

            def cbody(c, _, samp=samp, wav=wav, wbv=wbv, wcv=wcv, wdv=wdv):
                ch = jnp.full((16,), 0, jnp.int32) + c
                va = plsc.load_gather(ra, [samp, ch])
                vb = plsc.load_gather(rb, [samp, ch])
                vc = plsc.load_gather(rc, [samp, ch])
                vd = plsc.load_gather(rd, [samp, ch])
                acc = wav * va + wbv * vb + wcv * vc + wdv * vd
                plsc.store_scatter(ov, [samp, ch], acc)
                return 0

            lax.fori_loop(0, _C, cbody, 0, unroll=8)
        pltpu.sync_copy(ov, outr.at[pl.ds(s0, _K)])
        return carry

    lax.fori_loop(0, pw // _K, chunk, 0)


def kernel(inputs, grid):
    B, H, W, C = inputs.shape
    table = inputs.reshape(_N, _C)
    gx = grid[..., 0].reshape(_N)
    gy = grid[..., 1].reshape(_N)
    mesh = plsc.VectorSubcoreMesh(core_axis_name="c", subcore_axis_name="s")
    sample = functools.partial(
        pl.kernel,
        mesh=mesh,
        compiler_params=pltpu.CompilerParams(
            needs_layout_passes=False, use_tc_tiling_on_sc=False),
        out_type=jax.ShapeDtypeStruct((_N, _C), jnp.float32),
        scratch_types=[
            pltpu.VMEM((_K,), jnp.float32),
            pltpu.VMEM((_K,), jnp.float32),
            pltpu.VMEM((_K,), jnp.int32),
            pltpu.VMEM((_K,), jnp.int32),
            pltpu.VMEM((_K,), jnp.int32),
            pltpu.VMEM((_K,), jnp.int32),
            pltpu.VMEM((_K,), jnp.float32),
            pltpu.VMEM((_K,), jnp.float32),
            pltpu.VMEM((_K,), jnp.float32),
            pltpu.VMEM((_K,), jnp.float32),
            pltpu.VMEM((_K, _C), jnp.float32),
            pltpu.VMEM((_K, _C), jnp.float32),
            pltpu.VMEM((_K, _C), jnp.float32),
            pltpu.VMEM((_K, _C), jnp.float32),
            pltpu.VMEM((_K, _C), jnp.float32),
            pltpu.SemaphoreType.DMA,
        ],
    )(_sc_body)
    out = sample(table, gx, gy)
    return out.reshape(B, H, W, C)


# speedup vs baseline: 2.6088x; 2.6088x over previous
import functools

import jax
import jax.numpy as jnp
from jax import lax
from jax.experimental import pallas as pl
from jax.experimental.pallas import tpu as pltpu
from jax.experimental.pallas import tpu_sc as plsc

_B, _H, _W, _C = 4, 384, 384, 96
_N = _B * _H * _W
_HW = _H * _W
_K = 128
_G = _K // 16


def _floorf(x):
    xi = x.astype(jnp.int32)
    xf = xi.astype(jnp.float32)
    return jnp.where(xf > x, xf - 1.0, xf)


def _coords(g, extent):
    t = ((g + 1.0) * float(extent) - 1.0) * 0.5
    t1 = _floorf(t)
    f = t - t1
    snap = (1.0 - f) < 1e-5
    t1 = jnp.where(snap, t1 + 1.0, t1)
    f = jnp.where(snap, 0.0, f)
    return t1.astype(jnp.int32), f


def _sc_body(table, gxr, gyr, outr,
             gxv, gyv, ia, ib, ic, id_, war, wbr, wcr, wdr,
             ra, rb, rc, rd, ov, sem):
    nc = lax.axis_size("c")
    wid = lax.axis_index("s") * nc + lax.axis_index("c")
    pw = _N // (nc * lax.axis_size("s"))
    base = wid * pw
    brow = (base // _HW) * _HW
    iota = lax.iota(jnp.int32, 16)

    def chunk(t, carry):
        s0 = base + t * _K
        pltpu.sync_copy(gxr.at[pl.ds(s0, _K)], gxv)
        pltpu.sync_copy(gyr.at[pl.ds(s0, _K)], gyv)
        for g in range(_G):
            sl = pl.ds(g * 16, 16)
            x1, fx = _coords(gxv[sl], _W)
            y1, fy = _coords(gyv[sl], _H)
            x2 = x1 + 1
            y2 = y1 + 1
            vx1 = (x1 >= 0) & (x1 < _W)
            vx2 = (x2 >= 0) & (x2 < _W)
            vy1 = (y1 >= 0) & (y1 < _H)
            vy2 = (y2 >= 0) & (y2 < _H)
            cx1 = jnp.minimum(jnp.maximum(x1, 0), _W - 1)
            cx2 = jnp.minimum(jnp.maximum(x2, 0), _W - 1)
            cy1 = jnp.minimum(jnp.maximum(y1, 0), _H - 1)
            cy2 = jnp.minimum(jnp.maximum(y2, 0), _H - 1)
            ia[sl] = brow + cy1 * _W + cx1
            ib[sl] = brow + cy2 * _W + cx1
            ic[sl] = brow + cy1 * _W + cx2
            id_[sl] = brow + cy2 * _W + cx2
            gx1 = 1.0 - fx
            gy1 = 1.0 - fy
            war[sl] = jnp.where(vx1 & vy1, gx1 * gy1, 0.0)
            wbr[sl] = jnp.where(vx1 & vy2, gx1 * fy, 0.0)
            wcr[sl] = jnp.where(vx2 & vy1, fx * gy1, 0.0)
            wdr[sl] = jnp.where(vx2 & vy2, fx * fy, 0.0)
        da = pltpu.async_copy(table.at[ia], ra, sem)
        db = pltpu.async_copy(table.at[ib], rb, sem)
        dc = pltpu.async_copy(table.at[ic], rc, sem)
        dd = pltpu.async_copy(table.at[id_], rd, sem)
        da.wait()
        db.wait()
        dc.wait()
        dd.wait()
        for g in range(_G):
            sl = pl.ds(g * 16, 16)
            samp = iota + g * 16
            wav = war[sl]
            wbv = wbr[sl]
            wcv = wcr[sl]
            wdv = wdr[sl]

            def cbody(c, ch, samp=samp, wav=wav, wbv=wbv, wcv=wcv, wdv=wdv):
                va = plsc.load_gather(ra, [samp, ch])
                vb = plsc.load_gather(rb, [samp, ch])
                vc = plsc.load_gather(rc, [samp, ch])
                vd = plsc.load_gather(rd, [samp, ch])
                acc = wav * va + wbv * vb + wcv * vc + wdv * vd
                plsc.store_scatter(ov, [samp, ch], acc)
                ch = ch + 1
                return jnp.where(ch >= _C, ch - _C, ch)

            lax.fori_loop(0, _C, cbody, iota, unroll=8)
        pltpu.sync_copy(ov, outr.at[pl.ds(s0, _K)])
        return carry

    lax.fori_loop(0, pw // _K, chunk, 0)


def kernel(inputs, grid):
    B, H, W, C = inputs.shape
    table = inputs.reshape(_N, _C)
    gx = grid[..., 0].reshape(_N)
    gy = grid[..., 1].reshape(_N)
    mesh = plsc.VectorSubcoreMesh(core_axis_name="c", subcore_axis_name="s")
    sample = functools.partial(
        pl.kernel,
        mesh=mesh,
        compiler_params=pltpu.CompilerParams(
            needs_layout_passes=False, use_tc_tiling_on_sc=False),
        out_type=jax.ShapeDtypeStruct((_N, _C), jnp.float32),
        scratch_types=[
            pltpu.VMEM((_K,), jnp.float32),
            pltpu.VMEM((_K,), jnp.float32),
            pltpu.VMEM((_K,), jnp.int32),
            pltpu.VMEM((_K,), jnp.int32),
            pltpu.VMEM((_K,), jnp.int32),
            pltpu.VMEM((_K,), jnp.int32),
            pltpu.VMEM((_K,), jnp.float32),
            pltpu.VMEM((_K,), jnp.float32),
            pltpu.VMEM((_K,), jnp.float32),
            pltpu.VMEM((_K,), jnp.float32),
            pltpu.VMEM((_K, _C), jnp.float32),
            pltpu.VMEM((_K, _C), jnp.float32),
            pltpu.VMEM((_K, _C), jnp.float32),
            pltpu.VMEM((_K, _C), jnp.float32),
            pltpu.VMEM((_K, _C), jnp.float32),
            pltpu.SemaphoreType.DMA,
        ],
    )(_sc_body)
    out = sample(table, gx, gy)
    return out.reshape(B, H, W, C)


# speedup vs baseline: 2.8683x; 1.0995x over previous
import functools

import jax
import jax.numpy as jnp
from jax import lax
from jax.experimental import pallas as pl
from jax.experimental.pallas import tpu as pltpu
from jax.experimental.pallas import tpu_sc as plsc

_B, _H, _W, _C = 4, 384, 384, 96
_N = _B * _H * _W
_HW = _H * _W
_K = 64
_G = _K // 16


def _floorf(x):
    xi = x.astype(jnp.int32)
    xf = xi.astype(jnp.float32)
    return jnp.where(xf > x, xf - 1.0, xf)


def _coords(g, extent):
    t = ((g + 1.0) * float(extent) - 1.0) * 0.5
    t1 = _floorf(t)
    f = t - t1
    snap = (1.0 - f) < 1e-5
    t1 = jnp.where(snap, t1 + 1.0, t1)
    f = jnp.where(snap, 0.0, f)
    return t1.astype(jnp.int32), f


def _sc_body(table, gxr, gyr, outr, *s):
    set0, set1 = s[0:14], s[14:28]
    ov = s[28]
    sems = (s[29], s[30])
    nc = lax.axis_size("c")
    wid = lax.axis_index("s") * nc + lax.axis_index("c")
    pw = _N // (nc * lax.axis_size("s"))
    nt = pw // _K
    base = wid * pw
    brow = (base // _HW) * _HW
    iota = lax.iota(jnp.int32, 16)

    def phase_a(t, st, sem):
        gxv, gyv, ia, ib, ic, id_, war, wbr, wcr, wdr, ra, rb, rc, rd = st
        s0 = base + t * _K
        pltpu.sync_copy(gxr.at[pl.ds(s0, _K)], gxv)
        pltpu.sync_copy(gyr.at[pl.ds(s0, _K)], gyv)
        for g in range(_G):
            sl = pl.ds(g * 16, 16)
            x1, fx = _coords(gxv[sl], _W)
            y1, fy = _coords(gyv[sl], _H)
            x2 = x1 + 1
            y2 = y1 + 1
            vx1 = (x1 >= 0) & (x1 < _W)
            vx2 = (x2 >= 0) & (x2 < _W)
            vy1 = (y1 >= 0) & (y1 < _H)
            vy2 = (y2 >= 0) & (y2 < _H)
            cx1 = jnp.minimum(jnp.maximum(x1, 0), _W - 1)
            cx2 = jnp.minimum(jnp.maximum(x2, 0), _W - 1)
            cy1 = jnp.minimum(jnp.maximum(y1, 0), _H - 1)
            cy2 = jnp.minimum(jnp.maximum(y2, 0), _H - 1)
            ia[sl] = brow + cy1 * _W + cx1
            ib[sl] = brow + cy2 * _W + cx1
            ic[sl] = brow + cy1 * _W + cx2
            id_[sl] = brow + cy2 * _W + cx2
            gx1 = 1.0 - fx
            gy1 = 1.0 - fy
            war[sl] = jnp.where(vx1 & vy1, gx1 * gy1, 0.0)
            wbr[sl] = jnp.where(vx1 & vy2, gx1 * fy, 0.0)
            wcr[sl] = jnp.where(vx2 & vy1, fx * gy1, 0.0)
            wdr[sl] = jnp.where(vx2 & vy2, fx * fy, 0.0)
        pltpu.async_copy(table.at[ia], ra, sem)
        pltpu.async_copy(table.at[ib], rb, sem)
        pltpu.async_copy(table.at[ic], rc, sem)
        pltpu.async_copy(table.at[id_], rd, sem)

    def wait_gathers(st, sem):
        _, _, ia, ib, ic, id_, _, _, _, _, ra, rb, rc, rd = st
        pltpu.make_async_copy(table.at[ia], ra, sem).wait()
        pltpu.make_async_copy(table.at[ib], rb, sem).wait()
        pltpu.make_async_copy(table.at[ic], rc, sem).wait()
        pltpu.make_async_copy(table.at[id_], rd, sem).wait()

    def blend(t, st):
        _, _, _, _, _, _, war, wbr, wcr, wdr, ra, rb, rc, rd = st
        for g in range(_G):
            sl = pl.ds(g * 16, 16)
            samp = iota + g * 16
            wav = war[sl]
            wbv = wbr[sl]
            wcv = wcr[sl]
            wdv = wdr[sl]

            def cbody(c, ch, samp=samp, wav=wav, wbv=wbv, wcv=wcv, wdv=wdv):
                va = plsc.load_gather(ra, [samp, ch])
                vb = plsc.load_gather(rb, [samp, ch])
                vc = plsc.load_gather(rc, [samp, ch])
                vd = plsc.load_gather(rd, [samp, ch])
                acc = wav * va + wbv * vb + wcv * vc + wdv * vd
                plsc.store_scatter(ov, [samp, ch], acc)
                ch = ch + 1
                return jnp.where(ch >= _C, ch - _C, ch)

            lax.fori_loop(0, _C, cbody, iota, unroll=8)
        pltpu.sync_copy(ov, outr.at[pl.ds(base + t * _K, _K)])

    def step(t, cur, nxt, sem_cur, sem_nxt):
        phase_a(t + 1, nxt, sem_nxt)
        wait_gathers(cur, sem_cur)
        blend(t, cur)

    phase_a(0, set0, sems[0])

    def pair(u, carry):
        t = u * 2
        step(t, set0, set1, sems[0], sems[1])
        step(t + 1, set1, set0, sems[1], sems[0])
        return carry

    lax.fori_loop(0, nt // 2 - 1, pair, 0)
    step(nt - 2, set0, set1, sems[0], sems[1])
    wait_gathers(set1, sems[1])
    blend(nt - 1, set1)


def _chunk_set_types():
    return [
        pltpu.VMEM((_K,), jnp.float32),
        pltpu.VMEM((_K,), jnp.float32),
        pltpu.VMEM((_K,), jnp.int32),
        pltpu.VMEM((_K,), jnp.int32),
        pltpu.VMEM((_K,), jnp.int32),
        pltpu.VMEM((_K,), jnp.int32),
        pltpu.VMEM((_K,), jnp.float32),
        pltpu.VMEM((_K,), jnp.float32),
        pltpu.VMEM((_K,), jnp.float32),
        pltpu.VMEM((_K,), jnp.float32),
        pltpu.VMEM((_K, _C), jnp.float32),
        pltpu.VMEM((_K, _C), jnp.float32),
        pltpu.VMEM((_K, _C), jnp.float32),
        pltpu.VMEM((_K, _C), jnp.float32),
    ]


def kernel(inputs, grid):
    B, H, W, C = inputs.shape
    table = inputs.reshape(_N, _C)
    gx = grid[..., 0].reshape(_N)
    gy = grid[..., 1].reshape(_N)
    mesh = plsc.VectorSubcoreMesh(core_axis_name="c", subcore_axis_name="s")
    sample = functools.partial(
        pl.kernel,
        mesh=mesh,
        compiler_params=pltpu.CompilerParams(
            needs_layout_passes=False, use_tc_tiling_on_sc=False),
        out_type=jax.ShapeDtypeStruct((_N, _C), jnp.float32),
        scratch_types=(
            _chunk_set_types() + _chunk_set_types()
            + [pltpu.VMEM((_K, _C), jnp.float32),
               pltpu.SemaphoreType.DMA,
               pltpu.SemaphoreType.DMA]
        ),
    )(_sc_body)
    out = sample(table, gx, gy)
    return out.reshape(B, H, W, C)
